# baseline (speedup 1.0000x reference)
import functools

import jax
import jax.numpy as jnp
from jax.experimental import pallas as pl
from jax.experimental.pallas import tpu as pltpu

_N = 20000
_C = 128
_R = 160
_NPAD = _R * _C
_IMTOP = 200
_IOU_THR = 0.45
_SCORE_THR = 0.01
_NEG_INF = float("-inf")


def _comb(a, b):
    take_b = (b[0] > a[0]) | ((b[0] == a[0]) & (b[1] < a[1]))
    return (jnp.where(take_b, b[0], a[0]), jnp.where(take_b, b[1], a[1]))


def _argmax_vi(v, i):
    parts = [(v[k * 8:(k + 1) * 8, :], i[k * 8:(k + 1) * 8, :])
             for k in range(_R // 8)]
    while len(parts) > 1:
        nxt = [_comb(parts[p], parts[p + 1])
               for p in range(0, len(parts) - 1, 2)]
        if len(parts) % 2:
            nxt.append(parts[-1])
        parts = nxt
    cur = parts[0]
    for sh in (4, 2, 1):
        rolled = tuple(jnp.concatenate([arr[sh:, :], arr[:sh, :]], axis=0)
                       for arr in cur)
        cur = _comb(cur, rolled)
    cur = tuple(arr[0:1, :] for arr in cur)
    for sh in (64, 32, 16, 8, 4, 2, 1):
        rolled = tuple(jnp.concatenate([arr[:, sh:], arr[:, :sh]], axis=1)
                       for arr in cur)
        cur = _comb(cur, rolled)
    return cur


def _vmax11(x):
    return jnp.max(jnp.max(x, axis=0, keepdims=True), axis=1, keepdims=True)


def _nms_kernel(bxs_ref, sc_ref, out_ref, s_ref, a2_ref):
    s_raw = sc_ref[...]
    sv0 = jnp.where(s_raw > _SCORE_THR, s_raw, _NEG_INF)
    s_ref[...] = sv0

    row_i = jax.lax.broadcasted_iota(jnp.int32, (_R, _C), 0)
    col_i = jax.lax.broadcasted_iota(jnp.int32, (_R, _C), 1)
    idx_f = (row_i * _C + col_i).astype(jnp.float32)
    lane = jax.lax.broadcasted_iota(jnp.int32, (1, 128), 1)

    x1 = bxs_ref[0, :, :]
    y1 = bxs_ref[1, :, :]
    x2 = bxs_ref[2, :, :]
    y2 = bxs_ref[3, :, :]
    a2_ref[...] = (x2 - x1) * (y2 - y1)

    m_init, j_init = _argmax_vi(sv0, idx_f)

    def step(t, carry):
        m, j, j0, s0 = carry
        empty = m == _NEG_INF
        jj = jnp.where(empty, j0, j)
        onehot = idx_f == jj

        x1 = bxs_ref[0, :, :]
        y1 = bxs_ref[1, :, :]
        x2 = bxs_ref[2, :, :]
        y2 = bxs_ref[3, :, :]

        bx1 = _vmax11(jnp.where(onehot, x1, _NEG_INF))
        by1 = _vmax11(jnp.where(onehot, y1, _NEG_INF))
        bx2 = _vmax11(jnp.where(onehot, x2, _NEG_INF))
        by2 = _vmax11(jnp.where(onehot, y2, _NEG_INF))

        xx1 = jnp.maximum(bx1, x1)
        yy1 = jnp.maximum(by1, y1)
        xx2 = jnp.minimum(bx2, x2)
        yy2 = jnp.minimum(by2, y2)
        inter = jnp.maximum(xx2 - xx1, 0.0) * jnp.maximum(yy2 - yy1, 0.0)
        a1 = (bx2 - bx1) * (by2 - by1)
        iou = inter / (a1 + a2_ref[...] - inter + 1e-9)

        sv = s_ref[...]
        s_new = jnp.where((iou > _IOU_THR) | onehot, _NEG_INF, sv)
        s_ref[...] = s_new

        nm, nj = _argmax_vi(s_new, idx_f)

        sel_score = jnp.where(empty, s0, m)
        row = jnp.zeros((1, 128), jnp.float32)
        row = jnp.where(lane == 0, bx1, row)
        row = jnp.where(lane == 1, by1, row)
        row = jnp.where(lane == 2, bx2, row)
        row = jnp.where(lane == 3, by2, row)
        row = jnp.where(lane == 4, sel_score, row)
        out_ref[pl.ds(t, 1), :] = row

        first = t == 0
        j0 = jnp.where(first, jj, j0)
        s0 = jnp.where(first, sel_score, s0)
        return nm, nj, j0, s0

    def body(u, carry):
        carry = step(u * 2, carry)
        return step(u * 2 + 1, carry)

    jax.lax.fori_loop(
        0, _IMTOP // 2, body,
        (m_init, j_init,
         jnp.zeros((1, _C), jnp.float32),
         jnp.full((1, _C), _NEG_INF, jnp.float32)))


@functools.partial(jax.jit, static_argnames=())
def _run(boxes, scores):
    bxs = jnp.pad(boxes.T, ((0, 0), (0, _NPAD - _N))).reshape(4, _R, _C)
    sc = jnp.pad(scores, (0, _NPAD - _N)).reshape(_R, _C)
    out = pl.pallas_call(
        _nms_kernel,
        out_shape=jax.ShapeDtypeStruct((_IMTOP, 128), jnp.float32),
        scratch_shapes=[pltpu.VMEM((_R, _C), jnp.float32),
                        pltpu.VMEM((_R, _C), jnp.float32)],
    )(bxs, sc)
    return out[:, :5]


def kernel(boxes, scores, imtop):
    del imtop
    return _run(boxes, scores)


# speedup vs baseline: 1.3439x; 1.3439x over previous
import functools

import jax
import jax.numpy as jnp
from jax.experimental import pallas as pl
from jax.experimental.pallas import tpu as pltpu

_N = 20000
_C = 128
_R = 160
_NPAD = _R * _C
_IMTOP = 200
_IOU_THR = 0.45
_SCORE_THR = 0.01
_NEG_INF = float("-inf")


def _nms_kernel(bxs_ref, sc_ref, out_ref, s_ref, a2_ref):
    s_raw = sc_ref[...]
    sv0 = jnp.where(s_raw > _SCORE_THR, s_raw, _NEG_INF)
    s_ref[...] = sv0

    row_i = jax.lax.broadcasted_iota(jnp.int32, (_R, _C), 0)
    col_i = jax.lax.broadcasted_iota(jnp.int32, (_R, _C), 1)
    idx = row_i * _C + col_i
    lane = jax.lax.broadcasted_iota(jnp.int32, (1, 128), 1)

    x1 = bxs_ref[0, :, :]
    y1 = bxs_ref[1, :, :]
    x2 = bxs_ref[2, :, :]
    y2 = bxs_ref[3, :, :]
    a2_ref[...] = (x2 - x1) * (y2 - y1)

    m_init = jnp.max(sv0)
    j_init = jnp.min(jnp.where(sv0 == m_init, idx, _NPAD))

    def body(t, carry):
        m, j, j0, s0 = carry
        empty = m == _NEG_INF
        jj = jnp.where(empty, j0, j)
        onehot = idx == jj

        jr = jj // _C
        jc = jj % _C
        lmask = lane == jc

        def pick(c):
            rowv = bxs_ref[c, pl.ds(jr, 1), :]
            return jnp.max(jnp.where(lmask, rowv, _NEG_INF))

        bx1 = pick(0)
        by1 = pick(1)
        bx2 = pick(2)
        by2 = pick(3)

        x1 = bxs_ref[0, :, :]
        y1 = bxs_ref[1, :, :]
        x2 = bxs_ref[2, :, :]
        y2 = bxs_ref[3, :, :]

        xx1 = jnp.maximum(bx1, x1)
        yy1 = jnp.maximum(by1, y1)
        xx2 = jnp.minimum(bx2, x2)
        yy2 = jnp.minimum(by2, y2)
        inter = jnp.maximum(xx2 - xx1, 0.0) * jnp.maximum(yy2 - yy1, 0.0)
        a1 = (bx2 - bx1) * (by2 - by1)
        iou = inter / (a1 + a2_ref[...] - inter + 1e-9)

        sv = s_ref[...]
        s_new = jnp.where((iou > _IOU_THR) | onehot, _NEG_INF, sv)
        s_ref[...] = s_new

        m2 = jnp.max(s_new)
        j2 = jnp.min(jnp.where(s_new == m2, idx, _NPAD))

        sel_score = jnp.where(empty, s0, m)
        row = jnp.zeros((1, 128), jnp.float32)
        row = jnp.where(lane == 0, bx1, row)
        row = jnp.where(lane == 1, by1, row)
        row = jnp.where(lane == 2, bx2, row)
        row = jnp.where(lane == 3, by2, row)
        row = jnp.where(lane == 4, sel_score, row)
        out_ref[pl.ds(t, 1), :] = row

        j0 = jnp.where(t == 0, jj, j0)
        s0 = jnp.where(t == 0, sel_score, s0)
        return m2, j2, j0, s0

    jax.lax.fori_loop(
        0, _IMTOP, body,
        (m_init, j_init, jnp.int32(0), jnp.float32(_NEG_INF)))


@functools.partial(jax.jit, static_argnames=())
def _run(boxes, scores):
    bxs = jnp.pad(boxes.T, ((0, 0), (0, _NPAD - _N))).reshape(4, _R, _C)
    sc = jnp.pad(scores, (0, _NPAD - _N)).reshape(_R, _C)
    out = pl.pallas_call(
        _nms_kernel,
        out_shape=jax.ShapeDtypeStruct((_IMTOP, 128), jnp.float32),
        scratch_shapes=[pltpu.VMEM((_R, _C), jnp.float32),
                        pltpu.VMEM((_R, _C), jnp.float32)],
    )(bxs, sc)
    return out[:, :5]


def kernel(boxes, scores, imtop):
    del imtop
    return _run(boxes, scores)


# speedup vs baseline: 1.4767x; 1.0988x over previous
import functools

import jax
import jax.numpy as jnp
from jax.experimental import pallas as pl
from jax.experimental.pallas import tpu as pltpu

_N = 20000
_C = 128
_R = 160
_NPAD = _R * _C
_IMTOP = 200
_IOU_THR = 0.45
_SCORE_THR = 0.01
_NEG_INF = float("-inf")


def _vmax11(x):
    return jnp.max(jnp.max(x, axis=0, keepdims=True), axis=1, keepdims=True)


def _vmin11(x):
    return jnp.min(jnp.min(x, axis=0, keepdims=True), axis=1, keepdims=True)


def _nms_kernel(bxs_ref, sc_ref, out_ref, s_ref, a2_ref):
    s_raw = sc_ref[...]
    sv0 = jnp.where(s_raw > _SCORE_THR, s_raw, _NEG_INF)
    s_ref[...] = sv0

    row_i = jax.lax.broadcasted_iota(jnp.int32, (_R, _C), 0)
    col_i = jax.lax.broadcasted_iota(jnp.int32, (_R, _C), 1)
    idx = row_i * _C + col_i
    lane = jax.lax.broadcasted_iota(jnp.int32, (1, 128), 1)

    x1_0 = bxs_ref[0, :, :]
    y1_0 = bxs_ref[1, :, :]
    x2_0 = bxs_ref[2, :, :]
    y2_0 = bxs_ref[3, :, :]
    a2_ref[...] = (x2_0 - x1_0) * (y2_0 - y1_0)

    def select_top(s_arr):
        x1 = bxs_ref[0, :, :]
        y1 = bxs_ref[1, :, :]
        x2 = bxs_ref[2, :, :]
        y2 = bxs_ref[3, :, :]
        m = _vmax11(s_arr)
        eqm = s_arr == m
        j = _vmin11(jnp.where(eqm, idx, _NPAD))
        cnt = jnp.sum(jnp.where(eqm, 1.0, 0.0))

        def exact(mask):
            return (_vmax11(jnp.where(mask, x1, _NEG_INF)),
                    _vmax11(jnp.where(mask, y1, _NEG_INF)),
                    _vmax11(jnp.where(mask, x2, _NEG_INF)),
                    _vmax11(jnp.where(mask, y2, _NEG_INF)))

        coords = jax.lax.cond(cnt > 1.0,
                              lambda: exact(idx == j),
                              lambda: exact(eqm))
        return (m, j) + coords

    sel_init = select_top(sv0)

    z11 = jnp.zeros((1, 1), jnp.float32)

    def body(t, carry):
        (m, j, cx1, cy1, cx2, cy2, j0, s0,
         b0x1, b0y1, b0x2, b0y2) = carry
        empty = m == _NEG_INF
        jj = jnp.where(empty, j0, j)
        ex1 = jnp.where(empty, b0x1, cx1)
        ey1 = jnp.where(empty, b0y1, cy1)
        ex2 = jnp.where(empty, b0x2, cx2)
        ey2 = jnp.where(empty, b0y2, cy2)

        x1 = bxs_ref[0, :, :]
        y1 = bxs_ref[1, :, :]
        x2 = bxs_ref[2, :, :]
        y2 = bxs_ref[3, :, :]

        xx1 = jnp.maximum(ex1, x1)
        yy1 = jnp.maximum(ey1, y1)
        xx2 = jnp.minimum(ex2, x2)
        yy2 = jnp.minimum(ey2, y2)
        inter = jnp.maximum(xx2 - xx1, 0.0) * jnp.maximum(yy2 - yy1, 0.0)
        a1 = (ex2 - ex1) * (ey2 - ey1)
        iou = inter / (a1 + a2_ref[...] - inter + 1e-9)

        sv = s_ref[...]
        s_new = jnp.where((iou > _IOU_THR) | (idx == jj), _NEG_INF, sv)
        s_ref[...] = s_new

        nxt = select_top(s_new)

        sel_score = jnp.where(empty, s0, m)
        row = jnp.zeros((1, 128), jnp.float32)
        row = jnp.where(lane == 0, ex1, row)
        row = jnp.where(lane == 1, ey1, row)
        row = jnp.where(lane == 2, ex2, row)
        row = jnp.where(lane == 3, ey2, row)
        row = jnp.where(lane == 4, sel_score, row)
        out_ref[pl.ds(t, 1), :] = row

        first = t == 0
        j0 = jnp.where(first, jj, j0)
        s0 = jnp.where(first, sel_score, s0)
        b0x1 = jnp.where(first, ex1, b0x1)
        b0y1 = jnp.where(first, ey1, b0y1)
        b0x2 = jnp.where(first, ex2, b0x2)
        b0y2 = jnp.where(first, ey2, b0y2)
        return nxt + (j0, s0, b0x1, b0y1, b0x2, b0y2)

    jax.lax.fori_loop(
        0, _IMTOP, body,
        sel_init + (jnp.zeros((1, 1), jnp.int32),
                    jnp.full((1, 1), _NEG_INF, jnp.float32),
                    z11, z11, z11, z11))


@functools.partial(jax.jit, static_argnames=())
def _run(boxes, scores):
    bxs = jnp.pad(boxes.T, ((0, 0), (0, _NPAD - _N))).reshape(4, _R, _C)
    sc = jnp.pad(scores, (0, _NPAD - _N)).reshape(_R, _C)
    out = pl.pallas_call(
        _nms_kernel,
        out_shape=jax.ShapeDtypeStruct((_IMTOP, 128), jnp.float32),
        scratch_shapes=[pltpu.VMEM((_R, _C), jnp.float32),
                        pltpu.VMEM((_R, _C), jnp.float32)],
    )(bxs, sc)
    return out[:, :5]


def kernel(boxes, scores, imtop):
    del imtop
    return _run(boxes, scores)


# speedup vs baseline: 1.5695x; 1.0629x over previous
import functools

import jax
import jax.numpy as jnp
from jax.experimental import pallas as pl
from jax.experimental.pallas import tpu as pltpu

_N = 20000
_C = 128
_R = 160
_NPAD = _R * _C
_IMTOP = 200
_IOU_THR = 0.45
_SCORE_THR = 0.01
_NEG_INF = float("-inf")


def _vmax11(x):
    return jnp.max(jnp.max(x, axis=0, keepdims=True), axis=1, keepdims=True)


def _vmin11(x):
    return jnp.min(jnp.min(x, axis=0, keepdims=True), axis=1, keepdims=True)


def _nms_kernel(bxs_ref, sc_ref, out_ref, s_ref, a2_ref):
    s_raw = sc_ref[...]
    sv0 = jnp.where(s_raw > _SCORE_THR, s_raw, _NEG_INF)
    s_ref[...] = sv0

    row_i = jax.lax.broadcasted_iota(jnp.int32, (_R, _C), 0)
    col_i = jax.lax.broadcasted_iota(jnp.int32, (_R, _C), 1)
    idx = row_i * _C + col_i
    lane = jax.lax.broadcasted_iota(jnp.int32, (1, 128), 1)

    x1_0 = bxs_ref[0, :, :]
    y1_0 = bxs_ref[1, :, :]
    x2_0 = bxs_ref[2, :, :]
    y2_0 = bxs_ref[3, :, :]
    a2_ref[...] = (x2_0 - x1_0) * (y2_0 - y1_0)

    m_init = _vmax11(sv0)
    j_init = _vmin11(jnp.where(sv0 == m_init, idx, _NPAD))

    def step(t, carry):
        m, j, j0, s0 = carry
        empty = m == _NEG_INF
        jj = jnp.where(empty, j0, j)
        onehot = idx == jj

        x1 = bxs_ref[0, :, :]
        y1 = bxs_ref[1, :, :]
        x2 = bxs_ref[2, :, :]
        y2 = bxs_ref[3, :, :]

        bx1 = _vmax11(jnp.where(onehot, x1, _NEG_INF))
        by1 = _vmax11(jnp.where(onehot, y1, _NEG_INF))
        bx2 = _vmax11(jnp.where(onehot, x2, _NEG_INF))
        by2 = _vmax11(jnp.where(onehot, y2, _NEG_INF))

        xx1 = jnp.maximum(bx1, x1)
        yy1 = jnp.maximum(by1, y1)
        xx2 = jnp.minimum(bx2, x2)
        yy2 = jnp.minimum(by2, y2)
        inter = jnp.maximum(xx2 - xx1, 0.0) * jnp.maximum(yy2 - yy1, 0.0)
        a1 = (bx2 - bx1) * (by2 - by1)
        iou = inter / (a1 + a2_ref[...] - inter + 1e-9)

        sv = s_ref[...]
        s_new = jnp.where((iou > _IOU_THR) | onehot, _NEG_INF, sv)
        s_ref[...] = s_new

        m2 = _vmax11(s_new)
        j2 = _vmin11(jnp.where(s_new == m2, idx, _NPAD))

        sel_score = jnp.where(empty, s0, m)
        row = jnp.zeros((1, 128), jnp.float32)
        row = jnp.where(lane == 0, bx1, row)
        row = jnp.where(lane == 1, by1, row)
        row = jnp.where(lane == 2, bx2, row)
        row = jnp.where(lane == 3, by2, row)
        row = jnp.where(lane == 4, sel_score, row)
        out_ref[pl.ds(t, 1), :] = row

        j0 = jnp.where(t == 0, jj, j0)
        s0 = jnp.where(t == 0, sel_score, s0)
        return m2, j2, j0, s0

    def body(u, carry):
        carry = step(u * 2, carry)
        return step(u * 2 + 1, carry)

    jax.lax.fori_loop(
        0, _IMTOP // 2, body,
        (m_init, j_init,
         jnp.zeros((1, 1), jnp.int32),
         jnp.full((1, 1), _NEG_INF, jnp.float32)))


@functools.partial(jax.jit, static_argnames=())
def _run(boxes, scores):
    bxs = jnp.pad(boxes.T, ((0, 0), (0, _NPAD - _N))).reshape(4, _R, _C)
    sc = jnp.pad(scores, (0, _NPAD - _N)).reshape(_R, _C)
    out = pl.pallas_call(
        _nms_kernel,
        out_shape=jax.ShapeDtypeStruct((_IMTOP, 128), jnp.float32),
        scratch_shapes=[pltpu.VMEM((_R, _C), jnp.float32),
                        pltpu.VMEM((_R, _C), jnp.float32)],
    )(bxs, sc)
    return out[:, :5]


def kernel(boxes, scores, imtop):
    del imtop
    return _run(boxes, scores)
